# baseline (speedup 1.0000x reference)
import jax
import jax.numpy as jnp
from jax import lax
from jax.experimental import pallas as pl
from jax.experimental.pallas import tpu as pltpu
from jax.experimental.pallas import tpu_sc as plsc

T = 2048
D = 1024
E = 8
H = 2048
C = 512
S = E * C
SENT = S
NW = 32
HALF_S = S // 2
HALF_E = E // 2
TOK_PER_W = T // NW


def _router_body(x_ref, wg_ref, logits_ref, s1_ref, s2_ref,
                 g1_ref, g2_ref, aux_ref):
    x = x_ref[...]
    wg = wg_ref[...]
    logits = jnp.dot(x, wg, preferred_element_type=jnp.float32)
    logits_ref[...] = logits

    lt = logits.T
    m = jnp.max(lt, axis=0, keepdims=True)
    ex = jnp.exp(lt - m)
    pt = ex / jnp.sum(ex, axis=0, keepdims=True)

    rowid = lax.broadcasted_iota(jnp.int32, (E, T), 0)
    g1 = jnp.max(pt, axis=0, keepdims=True)
    i1 = jnp.min(jnp.where(pt == g1, rowid, E), axis=0, keepdims=True)
    m1 = (rowid == i1).astype(jnp.float32)
    pt2 = pt * (1.0 - m1)
    g2 = jnp.max(pt2, axis=0, keepdims=True)
    i2 = jnp.min(jnp.where(pt2 == g2, rowid, E), axis=0, keepdims=True)
    m2 = (rowid == i2).astype(jnp.float32)

    denom = g1 + g2 + 1e-9
    g1n = g1 / denom
    g2n = g2 / denom

    def icumsum(a):
        s = a
        k = 1
        while k < T:
            z = jnp.zeros((E, k), jnp.float32)
            s = s + jnp.concatenate([z, s[:, : T - k]], axis=1)
            k *= 2
        return s

    cs1 = icumsum(m1)
    pos1 = cs1 - m1
    p1t = jnp.sum(pos1 * m1, axis=0, keepdims=True)
    cnt1 = cs1[:, T - 1 : T]
    cs2 = icumsum(m2)
    pos2 = cs2 - m2 + cnt1
    p2t = jnp.sum(pos2 * m2, axis=0, keepdims=True)

    keep1 = (p1t < C).astype(jnp.float32)
    keep2 = (p2t < C).astype(jnp.float32)

    slot1 = i1 * C + p1t.astype(jnp.int32)
    slot2 = i2 * C + p2t.astype(jnp.int32)
    s1_ref[...] = jnp.where(keep1 > 0.0, slot1, SENT)
    s2_ref[...] = jnp.where(keep2 > 0.0, slot2, SENT)
    g1_ref[...] = g1n * keep1
    g2_ref[...] = g2n * keep2

    f = jnp.sum(m1 * keep1, axis=1, keepdims=True) * (1.0 / T)
    p = jnp.sum(pt, axis=1, keepdims=True) * (1.0 / T)
    aux_ref[...] = (E * jnp.sum(f * p)).reshape(1, 1)


def _router(x, wg):
    return pl.pallas_call(
        _router_body,
        out_shape=(
            jax.ShapeDtypeStruct((T, E), jnp.float32),
            jax.ShapeDtypeStruct((1, T), jnp.int32),
            jax.ShapeDtypeStruct((1, T), jnp.int32),
            jax.ShapeDtypeStruct((1, T), jnp.float32),
            jax.ShapeDtypeStruct((1, T), jnp.float32),
            jax.ShapeDtypeStruct((1, 1), jnp.float32),
        ),
    )(x, wg)


def _dispatch_body(lo, x_hbm, s1_hbm, s2_hbm, g1_hbm, g2_hbm, ein_hbm,
                   gsc_hbm, s1_v, s2_v, g1_v, g2_v, tfs_v, gsc_v, ia_v, ib_v,
                   ra_v, rb_v, sga, sgb, swa, swb):
    c = lax.axis_index("c")
    sc = lax.axis_index("s")
    wid = sc * 2 + c

    cps = (pltpu.async_copy(s1_hbm.at[0], s1_v, sga),
           pltpu.async_copy(s2_hbm.at[0], s2_v, sgb),
           pltpu.async_copy(g1_hbm.at[0], g1_v, swa),
           pltpu.async_copy(g2_hbm.at[0], g2_v, swb))
    for cp in cps:
        cp.wait()

    iota = lax.iota(jnp.int32, 16)

    def make_scat(sl_ref, gg_ref):
        def scat_body(i, carry):
            sl = sl_ref[pl.ds(i * 16, 16)]
            gg = gg_ref[pl.ds(i * 16, 16)]
            tok = iota + i * 16
            msk = (sl >= lo) & (sl < lo + HALF_S)
            sl_c = jnp.where(msk, sl - lo, 0)
            plsc.store_scatter(tfs_v, [sl_c], tok, mask=msk)
            plsc.store_scatter(gsc_v, [sl_c], gg, mask=msk)
            return carry
        return scat_body

    lax.fori_loop(0, T // 16, make_scat(s1_v, g1_v), 0)
    lax.fori_loop(0, T // 16, make_scat(s2_v, g2_v), 0)

    base = wid * (HALF_S // NW)
    rows = (ra_v, rb_v)
    idxs = (ia_v, ib_v)
    gsems = (sga, sgb)
    wsems = (swa, swb)

    def load_idx(k):
        b = k % 2
        for j in range(2):
            v = tfs_v[pl.ds(base + k * 32 + j * 16, 16)]
            idxs[b][pl.ds(j * 16, 16)] = jnp.minimum(
                jnp.maximum(v, 0), T - 1)

    def start_gather(k):
        b = k % 2
        return pltpu.async_copy(x_hbm.at[idxs[b]], rows[b], gsems[b])

    def start_write(k):
        b = k % 2
        return pltpu.async_copy(
            rows[b], ein_hbm.at[pl.ds(base + k * 32, 32)], wsems[b])

    load_idx(0)
    g0 = start_gather(0)
    load_idx(1)
    g1 = start_gather(1)
    g0.wait()
    w0 = start_write(0)
    g1.wait()
    w1 = start_write(1)
    w0.wait()
    w1.wait()

    pltpu.sync_copy(gsc_v.at[pl.ds(base, HALF_S // NW)],
                    gsc_hbm.at[0, pl.ds(base, HALF_S // NW)])


def _dispatch(lo, x, s1, s2, g1, g2):
    mesh = plsc.VectorSubcoreMesh(core_axis_name="c", subcore_axis_name="s")
    return pl.kernel(
        lambda *refs: _dispatch_body(lo, *refs),
        out_type=(
            jax.ShapeDtypeStruct((HALF_S, D), jnp.float32),
            jax.ShapeDtypeStruct((1, HALF_S), jnp.float32),
        ),
        mesh=mesh,
        compiler_params=pltpu.CompilerParams(needs_layout_passes=False),
        scratch_types=[
            pltpu.VMEM((T,), jnp.int32),
            pltpu.VMEM((T,), jnp.int32),
            pltpu.VMEM((T,), jnp.float32),
            pltpu.VMEM((T,), jnp.float32),
            pltpu.VMEM((HALF_S,), jnp.int32),
            pltpu.VMEM((HALF_S,), jnp.float32),
            pltpu.VMEM((32,), jnp.int32),
            pltpu.VMEM((32,), jnp.int32),
            pltpu.VMEM((32, D), jnp.float32),
            pltpu.VMEM((32, D), jnp.float32),
            pltpu.SemaphoreType.DMA,
            pltpu.SemaphoreType.DMA,
            pltpu.SemaphoreType.DMA,
            pltpu.SemaphoreType.DMA,
        ],
        name=f"dispatch_{lo}",
    )(x, s1, s2, g1, g2)


def _ffn_compute(ein_ref, w1_ref, b1_ref, w2_ref, b2_ref, gsc_ref, out_ref):
    xb = ein_ref[...].astype(jnp.bfloat16)
    w1 = w1_ref[0].astype(jnp.bfloat16)
    h = jnp.dot(xb, w1, preferred_element_type=jnp.float32)
    h = jax.nn.gelu(h + b1_ref[0])
    w2 = w2_ref[0].astype(jnp.bfloat16)
    o = jnp.dot(h.astype(jnp.bfloat16), w2,
                preferred_element_type=jnp.float32)
    o = o + b2_ref[0]
    out_ref[...] = o * gsc_ref[...].T


def _ffn_first_body(ein_ref, w1_ref, b1_ref, w2_ref, b2_ref, gsc_ref,
                    out_ref):
    _ffn_compute(ein_ref, w1_ref, b1_ref, w2_ref, b2_ref, gsc_ref, out_ref)


def _ffn_second_body(eo_in_ref, ein_ref, w1_ref, b1_ref, w2_ref, b2_ref,
                     gsc_ref, out_ref):
    del eo_in_ref
    p = pl.program_id(0)

    @pl.when(p < HALF_E)
    def _compute():
        _ffn_compute(ein_ref, w1_ref, b1_ref, w2_ref, b2_ref, gsc_ref,
                     out_ref)

    @pl.when(p == HALF_E)
    def _zeros():
        out_ref[...] = jnp.zeros((C, D), jnp.float32)


def _ffn_first(ein, w1, b1, w2, b2, gsc):
    return pl.pallas_call(
        _ffn_first_body,
        grid=(HALF_E,),
        in_specs=[
            pl.BlockSpec((C, D), lambda e: (e, 0)),
            pl.BlockSpec((1, D, H), lambda e: (e, 0, 0)),
            pl.BlockSpec((1, 1, H), lambda e: (e, 0, 0)),
            pl.BlockSpec((1, H, D), lambda e: (e, 0, 0)),
            pl.BlockSpec((1, 1, D), lambda e: (e, 0, 0)),
            pl.BlockSpec((1, C), lambda e: (0, e)),
        ],
        out_specs=pl.BlockSpec((C, D), lambda e: (e, 0)),
        out_shape=jax.ShapeDtypeStruct((S + C, D), jnp.float32),
    )(ein, w1, b1.reshape(E, 1, H), w2, b2.reshape(E, 1, D), gsc)


def _ffn_second(eo, ein, w1, b1, w2, b2, gsc):
    hi = HALF_E
    return pl.pallas_call(
        _ffn_second_body,
        grid=(HALF_E + 1,),
        in_specs=[
            pl.BlockSpec(memory_space=pl.ANY),
            pl.BlockSpec((C, D), lambda p: (jnp.minimum(p, hi - 1), 0)),
            pl.BlockSpec((1, D, H),
                         lambda p: (jnp.minimum(p + hi, E - 1), 0, 0)),
            pl.BlockSpec((1, 1, H),
                         lambda p: (jnp.minimum(p + hi, E - 1), 0, 0)),
            pl.BlockSpec((1, H, D),
                         lambda p: (jnp.minimum(p + hi, E - 1), 0, 0)),
            pl.BlockSpec((1, 1, D),
                         lambda p: (jnp.minimum(p + hi, E - 1), 0, 0)),
            pl.BlockSpec((1, C), lambda p: (0, jnp.minimum(p, hi - 1))),
        ],
        out_specs=pl.BlockSpec((C, D), lambda p: (p + hi, 0)),
        out_shape=jax.ShapeDtypeStruct((S + C, D), jnp.float32),
        input_output_aliases={0: 0},
    )(eo, ein, w1, b1.reshape(E, 1, H), w2, b2.reshape(E, 1, D), gsc)


def _combine_body(eo_hbm, s1_hbm, s2_hbm, y_hbm, ia_v, ib_v,
                  ra_v, rb_v, rc_v, sga, sgb, swa, swb):
    c = lax.axis_index("c")
    sc = lax.axis_index("s")
    wid = sc * 2 + c
    tb = wid * TOK_PER_W

    def add_into(dst, src):
        def add_body(t, carry):
            for j in range(D // 16):
                dst[t, pl.ds(j * 16, 16)] = (
                    dst[t, pl.ds(j * 16, 16)] + src[t, pl.ds(j * 16, 16)])
            return carry
        lax.fori_loop(0, 32, add_body, 0)

    pltpu.sync_copy(s1_hbm.at[0, pl.ds(tb, 32)], ia_v)
    pltpu.sync_copy(s2_hbm.at[0, pl.ds(tb, 32)], ib_v)
    ga = pltpu.async_copy(eo_hbm.at[ia_v], ra_v, sga)
    gb = pltpu.async_copy(eo_hbm.at[ib_v], rb_v, sgb)
    ga.wait()
    gb.wait()
    add_into(ra_v, rb_v)
    wa = pltpu.async_copy(ra_v, y_hbm.at[pl.ds(tb, 32)], swa)

    pltpu.sync_copy(s1_hbm.at[0, pl.ds(tb + 32, 32)], ia_v)
    pltpu.sync_copy(s2_hbm.at[0, pl.ds(tb + 32, 32)], ib_v)
    gc = pltpu.async_copy(eo_hbm.at[ia_v], rc_v, sga)
    gb2 = pltpu.async_copy(eo_hbm.at[ib_v], rb_v, sgb)
    gc.wait()
    gb2.wait()
    add_into(rc_v, rb_v)
    wc = pltpu.async_copy(rc_v, y_hbm.at[pl.ds(tb + 32, 32)], swb)

    wa.wait()
    wc.wait()


def _combine(eo, s1, s2):
    mesh = plsc.VectorSubcoreMesh(core_axis_name="c", subcore_axis_name="s")
    return pl.kernel(
        _combine_body,
        out_type=jax.ShapeDtypeStruct((T, D), jnp.float32),
        mesh=mesh,
        compiler_params=pltpu.CompilerParams(needs_layout_passes=False),
        scratch_types=[
            pltpu.VMEM((32,), jnp.int32),
            pltpu.VMEM((32,), jnp.int32),
            pltpu.VMEM((32, D), jnp.float32),
            pltpu.VMEM((32, D), jnp.float32),
            pltpu.VMEM((32, D), jnp.float32),
            pltpu.SemaphoreType.DMA,
            pltpu.SemaphoreType.DMA,
            pltpu.SemaphoreType.DMA,
            pltpu.SemaphoreType.DMA,
        ],
    )(eo, s1, s2)


def kernel(x, Wg, W1, b1, W2, b2):
    logits, s1, s2, g1, g2, aux = _router(x, Wg)
    einA, gscA = _dispatch(0, x, s1, s2, g1, g2)
    einB, gscB = _dispatch(HALF_S, x, s1, s2, g1, g2)
    eoA = _ffn_first(einA, W1, b1, W2, b2, gscA)
    eo = _ffn_second(eoA, einB, W1, b1, W2, b2, gscB)
    y = _combine(eo, s1, s2)
    metrics = {"aux_loss": aux[0, 0], "router_logits": logits}
    return y, metrics


# speedup vs baseline: 1.0288x; 1.0288x over previous
import jax
import jax.numpy as jnp
from jax import lax
from jax.experimental import pallas as pl
from jax.experimental.pallas import tpu as pltpu
from jax.experimental.pallas import tpu_sc as plsc

T = 2048
D = 1024
E = 8
H = 2048
C = 512
S = E * C
SENT = S
NW = 32
TOK_PER_W = T // NW


def _router_body(x_ref, wg_ref, logits_ref, s1_ref, s2_ref,
                 g1_ref, g2_ref, aux_ref):
    x = x_ref[...]
    wg = wg_ref[...]
    logits = jnp.dot(x, wg, preferred_element_type=jnp.float32)
    logits_ref[...] = logits

    lt = logits.T
    m = jnp.max(lt, axis=0, keepdims=True)
    ex = jnp.exp(lt - m)
    pt = ex / jnp.sum(ex, axis=0, keepdims=True)

    rowid = lax.broadcasted_iota(jnp.int32, (E, T), 0)
    g1 = jnp.max(pt, axis=0, keepdims=True)
    i1 = jnp.min(jnp.where(pt == g1, rowid, E), axis=0, keepdims=True)
    m1 = (rowid == i1).astype(jnp.float32)
    pt2 = pt * (1.0 - m1)
    g2 = jnp.max(pt2, axis=0, keepdims=True)
    i2 = jnp.min(jnp.where(pt2 == g2, rowid, E), axis=0, keepdims=True)
    m2 = (rowid == i2).astype(jnp.float32)

    denom = g1 + g2 + 1e-9
    g1n = g1 / denom
    g2n = g2 / denom

    def icumsum(a):
        s = a
        k = 1
        while k < T:
            z = jnp.zeros((E, k), jnp.float32)
            s = s + jnp.concatenate([z, s[:, : T - k]], axis=1)
            k *= 2
        return s

    cs1 = icumsum(m1)
    pos1 = cs1 - m1
    p1t = jnp.sum(pos1 * m1, axis=0, keepdims=True)
    cnt1 = cs1[:, T - 1 : T]
    cs2 = icumsum(m2)
    pos2 = cs2 - m2 + cnt1
    p2t = jnp.sum(pos2 * m2, axis=0, keepdims=True)

    keep1 = (p1t < C).astype(jnp.float32)
    keep2 = (p2t < C).astype(jnp.float32)

    slot1 = i1 * C + p1t.astype(jnp.int32)
    slot2 = i2 * C + p2t.astype(jnp.int32)
    s1_ref[...] = jnp.where(keep1 > 0.0, slot1, SENT)
    s2_ref[...] = jnp.where(keep2 > 0.0, slot2, SENT)
    g1_ref[...] = g1n * keep1
    g2_ref[...] = g2n * keep2

    f = jnp.sum(m1 * keep1, axis=1, keepdims=True) * (1.0 / T)
    p = jnp.sum(pt, axis=1, keepdims=True) * (1.0 / T)
    aux_ref[...] = (E * jnp.sum(f * p)).reshape(1, 1)


def _router(x, wg):
    return pl.pallas_call(
        _router_body,
        out_shape=(
            jax.ShapeDtypeStruct((T, E), jnp.float32),
            jax.ShapeDtypeStruct((1, T), jnp.int32),
            jax.ShapeDtypeStruct((1, T), jnp.int32),
            jax.ShapeDtypeStruct((1, T), jnp.float32),
            jax.ShapeDtypeStruct((1, T), jnp.float32),
            jax.ShapeDtypeStruct((1, 1), jnp.float32),
        ),
    )(x, wg)


def _dispatch_body(x_hbm, s1_hbm, s2_hbm, g1_hbm, g2_hbm, ein_hbm,
                   gsc_hbm, s1_v, s2_v, g1_v, g2_v, tfs_v, gsc_v, ia_v, ib_v,
                   ra_v, rb_v, sga, sgb, swa, swb):
    c = lax.axis_index("c")
    sc = lax.axis_index("s")
    wid = sc * 2 + c

    cps = (pltpu.async_copy(s1_hbm.at[0], s1_v, sga),
           pltpu.async_copy(s2_hbm.at[0], s2_v, sgb),
           pltpu.async_copy(g1_hbm.at[0], g1_v, swa),
           pltpu.async_copy(g2_hbm.at[0], g2_v, swb))
    for cp in cps:
        cp.wait()

    iota = lax.iota(jnp.int32, 16)

    def make_scat(sl_ref, gg_ref):
        def scat_body(i, carry):
            sl = sl_ref[pl.ds(i * 16, 16)]
            gg = gg_ref[pl.ds(i * 16, 16)]
            tok = iota + i * 16
            msk = sl < SENT
            sl_c = jnp.where(msk, sl, 0)
            plsc.store_scatter(tfs_v, [sl_c], tok, mask=msk)
            plsc.store_scatter(gsc_v, [sl_c], gg, mask=msk)
            return carry
        return scat_body

    lax.fori_loop(0, T // 16, make_scat(s1_v, g1_v), 0)
    lax.fori_loop(0, T // 16, make_scat(s2_v, g2_v), 0)

    base = wid * (S // NW)
    rows = (ra_v, rb_v)
    idxs = (ia_v, ib_v)
    gsems = (sga, sgb)
    wsems = (swa, swb)

    def load_idx(k):
        b = k % 2
        for j in range(2):
            v = tfs_v[pl.ds(base + k * 32 + j * 16, 16)]
            idxs[b][pl.ds(j * 16, 16)] = jnp.minimum(
                jnp.maximum(v, 0), T - 1)

    def start_gather(k):
        b = k % 2
        return pltpu.async_copy(x_hbm.at[idxs[b]], rows[b], gsems[b])

    def start_write(k):
        b = k % 2
        return pltpu.async_copy(
            rows[b], ein_hbm.at[pl.ds(base + k * 32, 32)], wsems[b])

    load_idx(0)
    g0 = start_gather(0)
    load_idx(1)
    g1 = start_gather(1)
    g0.wait()
    w0 = start_write(0)
    g1.wait()
    w1 = start_write(1)
    w0.wait()
    load_idx(2)
    g2 = start_gather(2)
    w1.wait()
    load_idx(3)
    g3 = start_gather(3)
    g2.wait()
    w2 = start_write(2)
    g3.wait()
    w3 = start_write(3)
    w2.wait()
    w3.wait()

    pltpu.sync_copy(gsc_v.at[pl.ds(base, S // NW)],
                    gsc_hbm.at[0, pl.ds(base, S // NW)])


def _dispatch(x, s1, s2, g1, g2):
    mesh = plsc.VectorSubcoreMesh(core_axis_name="c", subcore_axis_name="s")
    return pl.kernel(
        _dispatch_body,
        out_type=(
            jax.ShapeDtypeStruct((S, D), jnp.float32),
            jax.ShapeDtypeStruct((1, S), jnp.float32),
        ),
        mesh=mesh,
        compiler_params=pltpu.CompilerParams(needs_layout_passes=False),
        scratch_types=[
            pltpu.VMEM((T,), jnp.int32),
            pltpu.VMEM((T,), jnp.int32),
            pltpu.VMEM((T,), jnp.float32),
            pltpu.VMEM((T,), jnp.float32),
            pltpu.VMEM((S,), jnp.int32),
            pltpu.VMEM((S,), jnp.float32),
            pltpu.VMEM((32,), jnp.int32),
            pltpu.VMEM((32,), jnp.int32),
            pltpu.VMEM((32, D), jnp.float32),
            pltpu.VMEM((32, D), jnp.float32),
            pltpu.SemaphoreType.DMA,
            pltpu.SemaphoreType.DMA,
            pltpu.SemaphoreType.DMA,
            pltpu.SemaphoreType.DMA,
        ],
    )(x, s1, s2, g1, g2)


def _ffn_body(ein_ref, w1_ref, b1_ref, w2_ref, b2_ref, gsc_ref, out_ref):
    e = pl.program_id(0)

    @pl.when(e < E)
    def _compute():
        xb = ein_ref[...].astype(jnp.bfloat16)
        w1 = w1_ref[0].astype(jnp.bfloat16)
        h = jnp.dot(xb, w1, preferred_element_type=jnp.float32)
        h = jax.nn.gelu(h + b1_ref[0])
        w2 = w2_ref[0].astype(jnp.bfloat16)
        o = jnp.dot(h.astype(jnp.bfloat16), w2,
                    preferred_element_type=jnp.float32)
        o = o + b2_ref[0]
        out_ref[...] = o * gsc_ref[...].T

    @pl.when(e == E)
    def _zeros():
        out_ref[...] = jnp.zeros((C, D), jnp.float32)


def _ffn(ein, w1, b1, w2, b2, gsc):
    return pl.pallas_call(
        _ffn_body,
        grid=(E + 1,),
        in_specs=[
            pl.BlockSpec((C, D), lambda e: (jnp.minimum(e, E - 1), 0)),
            pl.BlockSpec((1, D, H), lambda e: (jnp.minimum(e, E - 1), 0, 0)),
            pl.BlockSpec((1, 1, H), lambda e: (jnp.minimum(e, E - 1), 0, 0)),
            pl.BlockSpec((1, H, D), lambda e: (jnp.minimum(e, E - 1), 0, 0)),
            pl.BlockSpec((1, 1, D), lambda e: (jnp.minimum(e, E - 1), 0, 0)),
            pl.BlockSpec((1, C), lambda e: (0, jnp.minimum(e, E - 1))),
        ],
        out_specs=pl.BlockSpec((C, D), lambda e: (e, 0)),
        out_shape=jax.ShapeDtypeStruct((S + C, D), jnp.float32),
    )(ein, w1, b1.reshape(E, 1, H), w2, b2.reshape(E, 1, D), gsc)


def _combine_body(eo_hbm, s1_hbm, s2_hbm, y_hbm, ia_v, ib_v,
                  ra_v, rb_v, rc_v, sga, sgb, swa, swb):
    c = lax.axis_index("c")
    sc = lax.axis_index("s")
    wid = sc * 2 + c
    tb = wid * TOK_PER_W

    def add_into(dst, src):
        def add_body(t, carry):
            for j in range(D // 16):
                dst[t, pl.ds(j * 16, 16)] = (
                    dst[t, pl.ds(j * 16, 16)] + src[t, pl.ds(j * 16, 16)])
            return carry
        lax.fori_loop(0, 32, add_body, 0)

    pltpu.sync_copy(s1_hbm.at[0, pl.ds(tb, 32)], ia_v)
    pltpu.sync_copy(s2_hbm.at[0, pl.ds(tb, 32)], ib_v)
    ga = pltpu.async_copy(eo_hbm.at[ia_v], ra_v, sga)
    gb = pltpu.async_copy(eo_hbm.at[ib_v], rb_v, sgb)
    ga.wait()
    gb.wait()
    add_into(ra_v, rb_v)
    wa = pltpu.async_copy(ra_v, y_hbm.at[pl.ds(tb, 32)], swa)

    pltpu.sync_copy(s1_hbm.at[0, pl.ds(tb + 32, 32)], ia_v)
    pltpu.sync_copy(s2_hbm.at[0, pl.ds(tb + 32, 32)], ib_v)
    gc = pltpu.async_copy(eo_hbm.at[ia_v], rc_v, sga)
    gb2 = pltpu.async_copy(eo_hbm.at[ib_v], rb_v, sgb)
    gc.wait()
    gb2.wait()
    add_into(rc_v, rb_v)
    wc = pltpu.async_copy(rc_v, y_hbm.at[pl.ds(tb + 32, 32)], swb)

    wa.wait()
    wc.wait()


def _combine(eo, s1, s2):
    mesh = plsc.VectorSubcoreMesh(core_axis_name="c", subcore_axis_name="s")
    return pl.kernel(
        _combine_body,
        out_type=jax.ShapeDtypeStruct((T, D), jnp.float32),
        mesh=mesh,
        compiler_params=pltpu.CompilerParams(needs_layout_passes=False),
        scratch_types=[
            pltpu.VMEM((32,), jnp.int32),
            pltpu.VMEM((32,), jnp.int32),
            pltpu.VMEM((32, D), jnp.float32),
            pltpu.VMEM((32, D), jnp.float32),
            pltpu.VMEM((32, D), jnp.float32),
            pltpu.SemaphoreType.DMA,
            pltpu.SemaphoreType.DMA,
            pltpu.SemaphoreType.DMA,
            pltpu.SemaphoreType.DMA,
        ],
    )(eo, s1, s2)


def kernel(x, Wg, W1, b1, W2, b2):
    logits, s1, s2, g1, g2, aux = _router(x, Wg)
    ein, gsc = _dispatch(x, s1, s2, g1, g2)
    eo = _ffn(ein, W1, b1, W2, b2, gsc)
    y = _combine(eo, s1, s2)
    metrics = {"aux_loss": aux[0, 0], "router_logits": logits}
    return y, metrics
